# baseline (speedup 1.0000x reference)
import functools

import jax
import jax.numpy as jnp
from jax import lax
from jax.experimental import pallas as pl
from jax.experimental.pallas import tpu as pltpu
from jax.experimental.pallas import tpu_sc as plsc

N = 8388608
NUM_BINS = 100
EPS = 1e-06
PAD = 112
ETAB = 128
BIG = 2.0e30
NEGBIG = -1.0e30

NC = 2
NS = 16
NW = NC * NS
PER_TILE = N // NW
CHUNK = 4096
NCHUNK = PER_TILE // CHUNK
NPAIR = NCHUNK // 2
VPC = CHUNK // 16


def _shuffle(tmp, v, idx):
    tmp[pl.ds(0, 16)] = v
    return plsc.load_gather(tmp, [idx])


def _vmax16(tmp, v, lane):
    for sh in (8, 4, 2, 1):
        v = jnp.maximum(v, _shuffle(tmp, v, lane ^ sh))
    return v


def _vsum16(tmp, v, lane):
    for sh in (8, 4, 2, 1):
        v = v + _shuffle(tmp, v, lane ^ sh)
    return v


def _vcumsum16(tmp, v, lane):
    for sh in (1, 2, 4, 8):
        g = _shuffle(tmp, v, jnp.maximum(lane - sh, 0))
        v = v + jnp.where(lane >= sh, g, 0.0)
    return v


def _lastval(tmp, v, lane):
    return _shuffle(tmp, v, lane * 0 + 15)


def _build_edges(lv, E, tmp, lane):
    chunks = [lv[pl.ds(16 * k, 16)] for k in range(7)]
    m = chunks[0]
    for k in range(1, 7):
        m = jnp.maximum(m, chunks[k])
    mx = _vmax16(tmp, m, lane)
    ws = []
    tot = jnp.zeros((16,), jnp.float32)
    for k in range(7):
        w = jnp.exp(chunks[k] - mx)
        ws.append(w)
        tot = tot + w
    s = _vsum16(tmp, tot, lane)
    ws2 = []
    tot2 = jnp.zeros((16,), jnp.float32)
    for k in range(7):
        w2 = ws[k] / s
        if k < 6:
            w2 = w2 + EPS
        else:
            w2 = w2 + jnp.where(lane < 4, EPS, 0.0)
        ws2.append(w2)
        tot2 = tot2 + w2
    s2 = _vsum16(tmp, tot2, lane)
    big = jnp.full((16,), BIG, jnp.float32)
    for k in range(ETAB // 16):
        E[pl.ds(16 * k, 16)] = big
    E[pl.ds(0, 16)] = jnp.where(lane == 0, 0.0, BIG)
    run = jnp.zeros((16,), jnp.float32)
    for k in range(7):
        c = _vcumsum16(tmp, ws2[k] / s2, lane) + run
        if k < 6:
            plsc.store_scatter(E, [lane + (16 * k + 1)], c)
        else:
            plsc.store_scatter(E, [lane + (16 * k + 1)], c, mask=lane < 4)
        run = _lastval(tmp, c, lane)


def _body(u_hbm, il_hbm, ol_hbm, out_hbm,
          il_v, ol_v, e_in, e_out, s_v, t_v, tmp_v,
          ub0, ub1, ob0, ob1, isem0, isem1, osem0, osem1):
    wid = lax.axis_index("s") * NC + lax.axis_index("c")
    base = wid * PER_TILE
    lane = lax.iota(jnp.int32, 16)

    pltpu.sync_copy(il_hbm, il_v)
    pltpu.sync_copy(ol_hbm, ol_v)
    _build_edges(il_v, e_in, tmp_v, lane)
    _build_edges(ol_v, e_out, tmp_v, lane)
    for k in range(7):
        idx = lane + 16 * k
        ia = plsc.load_gather(e_in, [idx])
        ib = plsc.load_gather(e_in, [idx + 1])
        oa = plsc.load_gather(e_out, [idx])
        ob = plsc.load_gather(e_out, [idx + 1])
        sl = (ob - oa) / (ib - ia + EPS)
        s_v[pl.ds(16 * k, 16)] = sl
        t_v[pl.ds(16 * k, 16)] = oa - sl * ia

    e63v = plsc.load_gather(e_in, [lane * 0 + 63])
    e31v = plsc.load_gather(e_in, [lane * 0 + 31])
    e95v = plsc.load_gather(e_in, [lane * 0 + 95])

    def compute(ub, obuf):
        @plsc.parallel_loop(0, VPC, 1, unroll=16)
        def vbody(k):
            off = k * 16
            uv = ub[pl.ds(off, 16)]
            uc = jnp.minimum(jnp.maximum(uv, 0.0), 1.0)
            pos = jnp.where(e63v < uc, 64, 0)
            ev32 = jnp.where(pos > 0, e95v, e31v)
            pos = pos + jnp.where(ev32 < uc, 32, 0)
            for step in (16, 8, 4, 2, 1):
                ev = plsc.load_gather(e_in, [pos + (step - 1)])
                pos = pos + jnp.where(ev < uc, step, 0)
            b = jnp.minimum(pos, NUM_BINS - 1)
            sl = plsc.load_gather(s_v, [b])
            tt = plsc.load_gather(t_v, [b])
            obuf[pl.ds(off, 16)] = tt + sl * uv

    def in_slice(ch):
        return u_hbm.at[pl.ds(base + ch * CHUNK, CHUNK)]

    def out_slice(ch):
        return out_hbm.at[pl.ds(base + ch * CHUNK, CHUNK)]

    pltpu.async_copy(in_slice(0), ub0, isem0)

    def outer(i, carry):
        ch0 = i * 2
        ch1 = ch0 + 1
        pltpu.async_copy(in_slice(ch1), ub1, isem1)
        pltpu.make_async_copy(in_slice(ch0), ub0, isem0).wait()

        @pl.when(i > 0)
        def _():
            pltpu.make_async_copy(ob0, out_slice(ch0 - 2), osem0).wait()

        compute(ub0, ob0)
        pltpu.async_copy(ob0, out_slice(ch0), osem0)

        @pl.when(i < NPAIR - 1)
        def _():
            pltpu.async_copy(in_slice(ch1 + 1), ub0, isem0)

        pltpu.make_async_copy(in_slice(ch1), ub1, isem1).wait()

        @pl.when(i > 0)
        def _():
            pltpu.make_async_copy(ob1, out_slice(ch1 - 2), osem1).wait()

        compute(ub1, ob1)
        pltpu.async_copy(ob1, out_slice(ch1), osem1)
        return carry

    lax.fori_loop(0, NPAIR, outer, 0)
    pltpu.make_async_copy(ob0, out_slice(NCHUNK - 2), osem0).wait()
    pltpu.make_async_copy(ob1, out_slice(NCHUNK - 1), osem1).wait()


@functools.partial(jax.jit, static_argnums=())
def _warp(u, il_pad, ol_pad):
    f = functools.partial(
        pl.kernel,
        compiler_params=pltpu.CompilerParams(needs_layout_passes=False),
        out_type=jax.ShapeDtypeStruct((N,), jnp.float32),
        mesh=plsc.VectorSubcoreMesh(
            core_axis_name="c", subcore_axis_name="s", num_cores=NC),
        scratch_types=[
            pltpu.VMEM((PAD,), jnp.float32),
            pltpu.VMEM((PAD,), jnp.float32),
            pltpu.VMEM((ETAB,), jnp.float32),
            pltpu.VMEM((ETAB,), jnp.float32),
            pltpu.VMEM((PAD,), jnp.float32),
            pltpu.VMEM((PAD,), jnp.float32),
            pltpu.VMEM((16,), jnp.float32),
            pltpu.VMEM((CHUNK,), jnp.float32),
            pltpu.VMEM((CHUNK,), jnp.float32),
            pltpu.VMEM((CHUNK,), jnp.float32),
            pltpu.VMEM((CHUNK,), jnp.float32),
            pltpu.SemaphoreType.DMA,
            pltpu.SemaphoreType.DMA,
            pltpu.SemaphoreType.DMA,
            pltpu.SemaphoreType.DMA,
        ],
    )(_body)
    return f(u, il_pad, ol_pad)


def kernel(u, input_logits_ema, output_logits_ema):
    il = jnp.pad(input_logits_ema.astype(jnp.float32), (0, PAD - NUM_BINS),
                 constant_values=NEGBIG)
    ol = jnp.pad(output_logits_ema.astype(jnp.float32), (0, PAD - NUM_BINS),
                 constant_values=NEGBIG)
    return _warp(u, il, ol)


# speedup vs baseline: 2.4392x; 2.4392x over previous
import functools

import jax
import jax.numpy as jnp
from jax import lax
from jax.experimental import pallas as pl
from jax.experimental.pallas import tpu as pltpu
from jax.experimental.pallas import tpu_sc as plsc

N = 8388608
NUM_BINS = 100
EPS = 1e-06
PAD = 112
ETAB = 128
BIG = 2.0e30
NEGBIG = -1.0e30

NC = 2
NS = 16
NW = NC * NS
PER_TILE = N // NW
CHUNK = 4096
NCHUNK = PER_TILE // CHUNK
NPAIR = NCHUNK // 2
VPC = CHUNK // 16


def _shuffle(tmp, v, idx):
    tmp[pl.ds(0, 16)] = v
    return plsc.load_gather(tmp, [idx])


def _vmax16(tmp, v, lane):
    for sh in (8, 4, 2, 1):
        v = jnp.maximum(v, _shuffle(tmp, v, lane ^ sh))
    return v


def _vsum16(tmp, v, lane):
    for sh in (8, 4, 2, 1):
        v = v + _shuffle(tmp, v, lane ^ sh)
    return v


def _vcumsum16(tmp, v, lane):
    for sh in (1, 2, 4, 8):
        g = _shuffle(tmp, v, jnp.maximum(lane - sh, 0))
        v = v + jnp.where(lane >= sh, g, 0.0)
    return v


def _lastval(tmp, v, lane):
    return _shuffle(tmp, v, lane * 0 + 15)


def _build_edges(lv, E, tmp, lane):
    chunks = [lv[pl.ds(16 * k, 16)] for k in range(7)]
    m = chunks[0]
    for k in range(1, 7):
        m = jnp.maximum(m, chunks[k])
    mx = _vmax16(tmp, m, lane)
    ws = []
    tot = jnp.zeros((16,), jnp.float32)
    for k in range(7):
        w = jnp.exp(chunks[k] - mx)
        ws.append(w)
        tot = tot + w
    s = _vsum16(tmp, tot, lane)
    ws2 = []
    tot2 = jnp.zeros((16,), jnp.float32)
    for k in range(7):
        w2 = ws[k] / s
        if k < 6:
            w2 = w2 + EPS
        else:
            w2 = w2 + jnp.where(lane < 4, EPS, 0.0)
        ws2.append(w2)
        tot2 = tot2 + w2
    s2 = _vsum16(tmp, tot2, lane)
    big = jnp.full((16,), BIG, jnp.float32)
    for k in range(ETAB // 16):
        E[pl.ds(16 * k, 16)] = big
    E[pl.ds(0, 16)] = jnp.where(lane == 0, 0.0, BIG)
    run = jnp.zeros((16,), jnp.float32)
    for k in range(7):
        c = _vcumsum16(tmp, ws2[k] / s2, lane) + run
        if k < 6:
            plsc.store_scatter(E, [lane + (16 * k + 1)], c)
        else:
            plsc.store_scatter(E, [lane + (16 * k + 1)], c, mask=lane < 4)
        run = _lastval(tmp, c, lane)


def _body(u_hbm, il_hbm, ol_hbm, out_hbm,
          il_v, ol_v, e_in, e_out, s_v, t_v, tmp_v,
          e16, s16, t16,
          ub0, ub1, ob0, ob1, isem0, isem1, osem0, osem1):
    wid = lax.axis_index("s") * NC + lax.axis_index("c")
    base = wid * PER_TILE
    lane = lax.iota(jnp.int32, 16)

    pltpu.sync_copy(il_hbm, il_v)
    pltpu.sync_copy(ol_hbm, ol_v)
    _build_edges(il_v, e_in, tmp_v, lane)
    _build_edges(ol_v, e_out, tmp_v, lane)
    for k in range(7):
        idx = lane + 16 * k
        ia = plsc.load_gather(e_in, [idx])
        ib = plsc.load_gather(e_in, [idx + 1])
        oa = plsc.load_gather(e_out, [idx])
        ob = plsc.load_gather(e_out, [idx + 1])
        sl = (ob - oa) / (ib - ia + EPS)
        s_v[pl.ds(16 * k, 16)] = sl
        t_v[pl.ds(16 * k, 16)] = oa - sl * ia

    for j in range(ETAB):
        e16[pl.ds(16 * j, 16)] = plsc.load_gather(e_in, [lane * 0 + j])
    for j in range(PAD):
        s16[pl.ds(16 * j, 16)] = plsc.load_gather(s_v, [lane * 0 + j])
        t16[pl.ds(16 * j, 16)] = plsc.load_gather(t_v, [lane * 0 + j])

    e63v = plsc.load_gather(e_in, [lane * 0 + 63])
    e31v = plsc.load_gather(e_in, [lane * 0 + 31])
    e95v = plsc.load_gather(e_in, [lane * 0 + 95])
    lim = lane + 16 * (NUM_BINS - 1)

    def compute(ub, obuf):
        @plsc.parallel_loop(0, VPC, 1, unroll=8)
        def vbody(k):
            off = k * 16
            uv = ub[pl.ds(off, 16)]
            uc = jnp.minimum(jnp.maximum(uv, 0.0), 1.0)
            m64 = e63v < uc
            pos = lane + jnp.where(m64, 64 * 16, 0)
            ev32 = jnp.where(m64, e95v, e31v)
            pos = pos + jnp.where(ev32 < uc, 32 * 16, 0)
            for step in (16, 8, 4, 2, 1):
                ev = plsc.load_gather(e16, [pos + (step - 1) * 16])
                pos = pos + jnp.where(ev < uc, step * 16, 0)
            b = jnp.minimum(pos, lim)
            sl = plsc.load_gather(s16, [b])
            tt = plsc.load_gather(t16, [b])
            obuf[pl.ds(off, 16)] = tt + sl * uv

    def in_slice(ch):
        return u_hbm.at[pl.ds(base + ch * CHUNK, CHUNK)]

    def out_slice(ch):
        return out_hbm.at[pl.ds(base + ch * CHUNK, CHUNK)]

    pltpu.async_copy(in_slice(0), ub0, isem0)

    def outer(i, carry):
        ch0 = i * 2
        ch1 = ch0 + 1
        pltpu.async_copy(in_slice(ch1), ub1, isem1)
        pltpu.make_async_copy(in_slice(ch0), ub0, isem0).wait()

        @pl.when(i > 0)
        def _():
            pltpu.make_async_copy(ob0, out_slice(ch0 - 2), osem0).wait()

        compute(ub0, ob0)
        pltpu.async_copy(ob0, out_slice(ch0), osem0)

        @pl.when(i < NPAIR - 1)
        def _():
            pltpu.async_copy(in_slice(ch1 + 1), ub0, isem0)

        pltpu.make_async_copy(in_slice(ch1), ub1, isem1).wait()

        @pl.when(i > 0)
        def _():
            pltpu.make_async_copy(ob1, out_slice(ch1 - 2), osem1).wait()

        compute(ub1, ob1)
        pltpu.async_copy(ob1, out_slice(ch1), osem1)
        return carry

    lax.fori_loop(0, NPAIR, outer, 0)
    pltpu.make_async_copy(ob0, out_slice(NCHUNK - 2), osem0).wait()
    pltpu.make_async_copy(ob1, out_slice(NCHUNK - 1), osem1).wait()


@functools.partial(jax.jit, static_argnums=())
def _warp(u, il_pad, ol_pad):
    f = functools.partial(
        pl.kernel,
        compiler_params=pltpu.CompilerParams(needs_layout_passes=False),
        out_type=jax.ShapeDtypeStruct((N,), jnp.float32),
        mesh=plsc.VectorSubcoreMesh(
            core_axis_name="c", subcore_axis_name="s", num_cores=NC),
        scratch_types=[
            pltpu.VMEM((PAD,), jnp.float32),
            pltpu.VMEM((PAD,), jnp.float32),
            pltpu.VMEM((ETAB,), jnp.float32),
            pltpu.VMEM((ETAB,), jnp.float32),
            pltpu.VMEM((PAD,), jnp.float32),
            pltpu.VMEM((PAD,), jnp.float32),
            pltpu.VMEM((16,), jnp.float32),
            pltpu.VMEM((16 * ETAB,), jnp.float32),
            pltpu.VMEM((16 * PAD,), jnp.float32),
            pltpu.VMEM((16 * PAD,), jnp.float32),
            pltpu.VMEM((CHUNK,), jnp.float32),
            pltpu.VMEM((CHUNK,), jnp.float32),
            pltpu.VMEM((CHUNK,), jnp.float32),
            pltpu.VMEM((CHUNK,), jnp.float32),
            pltpu.SemaphoreType.DMA,
            pltpu.SemaphoreType.DMA,
            pltpu.SemaphoreType.DMA,
            pltpu.SemaphoreType.DMA,
        ],
    )(_body)
    return f(u, il_pad, ol_pad)


def kernel(u, input_logits_ema, output_logits_ema):
    il = jnp.pad(input_logits_ema.astype(jnp.float32), (0, PAD - NUM_BINS),
                 constant_values=NEGBIG)
    ol = jnp.pad(output_logits_ema.astype(jnp.float32), (0, PAD - NUM_BINS),
                 constant_values=NEGBIG)
    return _warp(u, il, ol)
